# Initial kernel scaffold; baseline (speedup 1.0000x reference)
import functools

import jax
import jax.numpy as jnp
from jax import lax
from jax.experimental import pallas as pl
from jax.experimental.pallas import tpu as pltpu
from jax.experimental.pallas import tpu_sc as plsc

N = 10000
E = 320000
C = 1000
H = 128
NCLS = 40

NSC = 2
NTILE = 16
NW = NSC * NTILE
RPT = N // NTILE
EPT = E // NW
KE = 80
NCHUNK = EPT // KE
BN = 1000
GRID = N // BN

_MESH = plsc.VectorSubcoreMesh(core_axis_name="c", subcore_axis_name="s")


@functools.partial(
    pl.kernel,
    out_type=jax.ShapeDtypeStruct((NSC, N, 16), jnp.float32),
    mesh=_MESH,
    scratch_types=[
        pltpu.VMEM((KE,), jnp.int32),
        pltpu.VMEM((KE, 16), jnp.float32),
        pltpu.VMEM((RPT, 16), jnp.float32),
        pltpu.VMEM_SHARED((N, 16), jnp.float32),
    ],
)
def _deg_kernel(adj_hbm, ones_hbm, zeros_hbm, out_hbm,
                dst_v, ones_v, strip_v, deg_sh):
    cid = lax.axis_index("c")
    sid = lax.axis_index("s")
    r0 = sid * RPT
    pltpu.sync_copy(zeros_hbm, strip_v)
    pltpu.sync_copy(strip_v, deg_sh.at[pl.ds(r0, RPT)])
    pltpu.sync_copy(ones_hbm, ones_v)
    plsc.subcore_barrier()
    ebase = cid * (E // NSC) + sid * EPT

    def body(i, carry):
        b = ebase + i * KE
        pltpu.sync_copy(adj_hbm.at[1, pl.ds(b, KE)], dst_v)
        pltpu.sync_copy(ones_v, deg_sh.at[dst_v], add=True)
        return carry

    lax.fori_loop(0, NCHUNK, body, 0)
    plsc.subcore_barrier()
    pltpu.sync_copy(deg_sh.at[pl.ds(r0, RPT)], strip_v)
    pltpu.sync_copy(strip_v, out_hbm.at[cid, pl.ds(r0, RPT)])


@functools.partial(
    pl.kernel,
    out_type=jax.ShapeDtypeStruct((NSC, N, H), jnp.float32),
    mesh=_MESH,
    scratch_types=[
        pltpu.VMEM((KE,), jnp.int32),
        pltpu.VMEM((KE,), jnp.int32),
        pltpu.VMEM((KE, H), jnp.float32),
        pltpu.VMEM((RPT, H), jnp.float32),
        pltpu.VMEM_SHARED((N, H), jnp.float32),
        pltpu.SemaphoreType.DMA,
    ],
)
def _segsum_kernel(xs_hbm, adj_hbm, out_hbm,
                   src_v, dst_v, rows_v, strip_v, acc_sh, sem):
    cid = lax.axis_index("c")
    sid = lax.axis_index("s")
    r0 = sid * RPT
    pltpu.sync_copy(xs_hbm.at[pl.ds(r0, RPT)], strip_v)
    pltpu.sync_copy(strip_v, acc_sh.at[pl.ds(r0, RPT)])
    plsc.subcore_barrier()
    ebase = cid * (E // NSC) + sid * EPT

    def body(i, carry):
        b = ebase + i * KE
        pltpu.sync_copy(adj_hbm.at[0, pl.ds(b, KE)], src_v)
        pltpu.sync_copy(adj_hbm.at[1, pl.ds(b, KE)], dst_v)
        pltpu.async_copy(xs_hbm.at[src_v], rows_v, sem).wait()
        pltpu.sync_copy(rows_v, acc_sh.at[dst_v], add=True)
        return carry

    lax.fori_loop(0, NCHUNK, body, 0)
    plsc.subcore_barrier()
    pltpu.sync_copy(acc_sh.at[pl.ds(r0, RPT)], strip_v)
    pltpu.sync_copy(strip_v, out_hbm.at[cid, pl.ds(r0, RPT)])


def _tcA_body(x_ref, w1_ref, degp_ref, xs1_ref, dinv8_ref):
    deg = degp_ref[0, :, 0:1] + degp_ref[1, :, 0:1] + 1.0
    dinv = lax.rsqrt(deg)
    xw = jnp.dot(x_ref[...], w1_ref[...], preferred_element_type=jnp.float32)
    xs1_ref[...] = dinv * xw
    dinv8_ref[...] = jnp.broadcast_to(dinv, (BN, 8))


_tcA = pl.pallas_call(
    _tcA_body,
    grid=(GRID,),
    in_specs=[
        pl.BlockSpec((BN, H), lambda i: (i, 0)),
        pl.BlockSpec((H, H), lambda i: (0, 0)),
        pl.BlockSpec((NSC, BN, 16), lambda i: (0, i, 0)),
    ],
    out_specs=[
        pl.BlockSpec((BN, H), lambda i: (i, 0)),
        pl.BlockSpec((BN, 8), lambda i: (i, 0)),
    ],
    out_shape=[
        jax.ShapeDtypeStruct((N, H), jnp.float32),
        jax.ShapeDtypeStruct((N, 8), jnp.float32),
    ],
)


def _tcB_body(acc_ref, xs1_ref, dinv8_ref, b1_ref, w2_ref, xs2_ref):
    dinv = dinv8_ref[:, 0:1]
    h1 = jnp.maximum(
        dinv * (acc_ref[0] + acc_ref[1] - xs1_ref[...]) + b1_ref[...], 0.0)
    xw2 = jnp.dot(h1, w2_ref[...], preferred_element_type=jnp.float32)
    xs2_ref[...] = dinv * xw2


_tcB = pl.pallas_call(
    _tcB_body,
    grid=(GRID,),
    in_specs=[
        pl.BlockSpec((NSC, BN, H), lambda i: (0, i, 0)),
        pl.BlockSpec((BN, H), lambda i: (i, 0)),
        pl.BlockSpec((BN, 8), lambda i: (i, 0)),
        pl.BlockSpec((H,), lambda i: (0,)),
        pl.BlockSpec((H, H), lambda i: (0, 0)),
    ],
    out_specs=pl.BlockSpec((BN, H), lambda i: (i, 0)),
    out_shape=jax.ShapeDtypeStruct((N, H), jnp.float32),
)


def _tcC_body(acc_ref, xs2_ref, dinv8_ref, b2_ref, comm_ref, map_ref,
              gw_ref, gb_ref, cw_ref, cb_ref, out_ref):
    dinv = dinv8_ref[:, 0:1]
    h2 = jnp.maximum(
        dinv * (acc_ref[0] + acc_ref[1] - xs2_ref[...]) + b2_ref[...], 0.0)
    m = map_ref[0, 0, :]
    onehot = (m[:, None] == lax.broadcasted_iota(jnp.int32, (1, C), 1)
              ).astype(jnp.float32)
    hc = jnp.dot(onehot, comm_ref[...], preferred_element_type=jnp.float32)
    gate = jax.nn.sigmoid(
        jnp.dot(h2, gw_ref[0:H, :], preferred_element_type=jnp.float32)
        + jnp.dot(hc, gw_ref[H:2 * H, :], preferred_element_type=jnp.float32)
        + gb_ref[...])
    hf = gate * h2 + (1.0 - gate) * hc
    logits = jnp.dot(hf, cw_ref[...], preferred_element_type=jnp.float32) \
        + cb_ref[...]
    mx = jnp.max(logits, axis=-1, keepdims=True)
    z = logits - mx
    lse = jnp.log(jnp.sum(jnp.exp(z), axis=-1, keepdims=True))
    out_ref[...] = z - lse


_tcC = pl.pallas_call(
    _tcC_body,
    grid=(GRID,),
    in_specs=[
        pl.BlockSpec((NSC, BN, H), lambda i: (0, i, 0)),
        pl.BlockSpec((BN, H), lambda i: (i, 0)),
        pl.BlockSpec((BN, 8), lambda i: (i, 0)),
        pl.BlockSpec((H,), lambda i: (0,)),
        pl.BlockSpec((C, H), lambda i: (0, 0)),
        pl.BlockSpec((1, 1, BN), lambda i: (i, 0, 0)),
        pl.BlockSpec((2 * H, 1), lambda i: (0, 0)),
        pl.BlockSpec((1,), lambda i: (0,)),
        pl.BlockSpec((H, NCLS), lambda i: (0, 0)),
        pl.BlockSpec((NCLS,), lambda i: (0,)),
    ],
    out_specs=pl.BlockSpec((BN, NCLS), lambda i: (i, 0)),
    out_shape=jax.ShapeDtypeStruct((N, NCLS), jnp.float32),
)


def kernel(node_features, node_adj, comm_features, comm_adj, node_to_comm_map,
           W1, b1, W2, b2, gate_W, gate_b, cls_W, cls_b):
    del comm_adj
    ones_c = jnp.ones((KE, 16), jnp.float32)
    zeros_c = jnp.zeros((RPT, 16), jnp.float32)
    map3 = node_to_comm_map.reshape(GRID, 1, BN)

    deg_p = _deg_kernel(node_adj, ones_c, zeros_c)
    xs1, dinv8 = _tcA(node_features, W1, deg_p)
    acc1 = _segsum_kernel(xs1, node_adj)
    xs2 = _tcB(acc1, xs1, dinv8, b1, W2)
    acc2 = _segsum_kernel(xs2, node_adj)
    out = _tcC(acc2, xs2, dinv8, b2, comm_features, map3,
               gate_W, gate_b, cls_W, cls_b)
    return out


# speedup vs baseline: 6.6039x; 6.6039x over previous
import functools

import jax
import jax.numpy as jnp
from jax import lax
from jax.experimental import pallas as pl
from jax.experimental.pallas import tpu as pltpu
from jax.experimental.pallas import tpu_sc as plsc

N = 10000
E = 320000
C = 1000
H = 128
NCLS = 40

NSC = 2
NTILE = 16
NP = 10240
SHALF = NP // 2
SACC = SHALF + 8
TRASH = SHALF
SRPT = SHALF // NTILE
EPT = E // NTILE
KE = 80
NCHUNK = EPT // KE
BN = 1000
GRID = N // BN

_MESH = plsc.VectorSubcoreMesh(core_axis_name="c", subcore_axis_name="s",
                               num_cores=NSC, num_subcores=NTILE)


def _localize(dst_v, loc_v, lo):
    for j in range(KE // 16):
        d = dst_v[pl.ds(j * 16, 16)]
        rel = d - lo
        owned = (rel >= 0) & (rel < SHALF)
        loc_v[pl.ds(j * 16, 16)] = jnp.where(owned, rel, TRASH)


@functools.partial(
    pl.kernel,
    out_type=jax.ShapeDtypeStruct((NP, H), jnp.float32),
    mesh=_MESH,
    scratch_types=[
        pltpu.VMEM((KE,), jnp.int32),
        pltpu.VMEM((KE,), jnp.int32),
        pltpu.VMEM((KE, H), jnp.float32),
        pltpu.VMEM((328, H), jnp.float32),
        pltpu.VMEM_SHARED((SACC, H), jnp.float32),
    ],
)
def _deg_kernel(dst_hbm, ones_hbm, zeros_hbm, out_hbm,
                dst_v, loc_v, ones_v, strip_v, deg_sh):
    cid = lax.axis_index("c")
    sid = lax.axis_index("s")
    lo = cid * SHALF
    r0 = sid * SRPT
    pltpu.sync_copy(zeros_hbm, strip_v)
    pltpu.sync_copy(strip_v, deg_sh.at[pl.ds(r0, 328)])
    pltpu.sync_copy(ones_hbm, ones_v)
    plsc.subcore_barrier()
    ebase = sid * EPT

    def body(i, carry):
        b = ebase + i * KE
        pltpu.sync_copy(dst_hbm.at[pl.ds(b, KE)], dst_v)
        _localize(dst_v, loc_v, lo)
        pltpu.sync_copy(ones_v, deg_sh.at[loc_v], add=True)
        return carry

    lax.fori_loop(0, NCHUNK, body, 0)
    plsc.subcore_barrier()
    pltpu.sync_copy(deg_sh.at[pl.ds(r0, SRPT)], strip_v.at[pl.ds(0, SRPT)])
    pltpu.sync_copy(strip_v.at[pl.ds(0, SRPT)],
                    out_hbm.at[pl.ds(lo + r0, SRPT)])


@functools.partial(
    pl.kernel,
    out_type=jax.ShapeDtypeStruct((NP, H), jnp.float32),
    mesh=_MESH,
    scratch_types=[
        pltpu.VMEM((KE,), jnp.int32),
        pltpu.VMEM((KE,), jnp.int32),
        pltpu.VMEM((KE,), jnp.int32),
        pltpu.VMEM((KE, H), jnp.float32),
        pltpu.VMEM((SRPT, H), jnp.float32),
        pltpu.VMEM_SHARED((SACC, H), jnp.float32),
        pltpu.SemaphoreType.DMA,
    ],
)
def _segsum_kernel(xs_hbm, src_hbm, dst_hbm, out_hbm,
                   src_v, dst_v, loc_v, rows_v, strip_v, acc_sh, sem):
    cid = lax.axis_index("c")
    sid = lax.axis_index("s")
    lo = cid * SHALF
    r0 = sid * SRPT
    pltpu.sync_copy(xs_hbm.at[pl.ds(lo + r0, SRPT)], strip_v)
    pltpu.sync_copy(strip_v, acc_sh.at[pl.ds(r0, SRPT)])
    plsc.subcore_barrier()
    ebase = sid * EPT

    def body(i, carry):
        b = ebase + i * KE
        pltpu.sync_copy(src_hbm.at[pl.ds(b, KE)], src_v)
        pltpu.sync_copy(dst_hbm.at[pl.ds(b, KE)], dst_v)
        _localize(dst_v, loc_v, lo)
        pltpu.async_copy(xs_hbm.at[src_v], rows_v, sem).wait()
        pltpu.sync_copy(rows_v, acc_sh.at[loc_v], add=True)
        return carry

    lax.fori_loop(0, NCHUNK, body, 0)
    plsc.subcore_barrier()
    pltpu.sync_copy(acc_sh.at[pl.ds(r0, SRPT)], strip_v)
    pltpu.sync_copy(strip_v, out_hbm.at[pl.ds(lo + r0, SRPT)])


def _tcA_body(x_ref, w1_ref, degp_ref, xs1_ref, dinv8_ref):
    deg = degp_ref[:, 0:1] + 1.0
    dinv = lax.rsqrt(deg)
    xw = jnp.dot(x_ref[...], w1_ref[...], preferred_element_type=jnp.float32)
    xs1_ref[...] = dinv * xw
    dinv8_ref[...] = jnp.broadcast_to(dinv, (BN, 8))


_tcA = pl.pallas_call(
    _tcA_body,
    grid=(GRID,),
    in_specs=[
        pl.BlockSpec((BN, H), lambda i: (i, 0)),
        pl.BlockSpec((H, H), lambda i: (0, 0)),
        pl.BlockSpec((BN, H), lambda i: (i, 0)),
    ],
    out_specs=[
        pl.BlockSpec((BN, H), lambda i: (i, 0)),
        pl.BlockSpec((BN, 8), lambda i: (i, 0)),
    ],
    out_shape=[
        jax.ShapeDtypeStruct((NP, H), jnp.float32),
        jax.ShapeDtypeStruct((N, 8), jnp.float32),
    ],
)


def _tcB_body(acc_ref, dinv8_ref, b1_ref, w2_ref, xs2_ref):
    dinv = dinv8_ref[:, 0:1]
    h1 = jnp.maximum(dinv * acc_ref[...] + b1_ref[...], 0.0)
    xw2 = jnp.dot(h1, w2_ref[...], preferred_element_type=jnp.float32)
    xs2_ref[...] = dinv * xw2


_tcB = pl.pallas_call(
    _tcB_body,
    grid=(GRID,),
    in_specs=[
        pl.BlockSpec((BN, H), lambda i: (i, 0)),
        pl.BlockSpec((BN, 8), lambda i: (i, 0)),
        pl.BlockSpec((H,), lambda i: (0,)),
        pl.BlockSpec((H, H), lambda i: (0, 0)),
    ],
    out_specs=pl.BlockSpec((BN, H), lambda i: (i, 0)),
    out_shape=jax.ShapeDtypeStruct((NP, H), jnp.float32),
)


def _tcC_body(acc_ref, dinv8_ref, b2_ref, comm_ref, map_ref,
              gw_ref, gb_ref, cw_ref, cb_ref, out_ref):
    dinv = dinv8_ref[:, 0:1]
    h2 = jnp.maximum(dinv * acc_ref[...] + b2_ref[...], 0.0)
    m = map_ref[0, 0, :]
    onehot = (m[:, None] == lax.broadcasted_iota(jnp.int32, (1, C), 1)
              ).astype(jnp.float32)
    hc = jnp.dot(onehot, comm_ref[...], preferred_element_type=jnp.float32)
    gate = jax.nn.sigmoid(
        jnp.dot(h2, gw_ref[0:H, :], preferred_element_type=jnp.float32)
        + jnp.dot(hc, gw_ref[H:2 * H, :], preferred_element_type=jnp.float32)
        + gb_ref[...])
    hf = gate * h2 + (1.0 - gate) * hc
    logits = jnp.dot(hf, cw_ref[...], preferred_element_type=jnp.float32) \
        + cb_ref[...]
    mx = jnp.max(logits, axis=-1, keepdims=True)
    z = logits - mx
    lse = jnp.log(jnp.sum(jnp.exp(z), axis=-1, keepdims=True))
    out_ref[...] = z - lse


_tcC = pl.pallas_call(
    _tcC_body,
    grid=(GRID,),
    in_specs=[
        pl.BlockSpec((BN, H), lambda i: (i, 0)),
        pl.BlockSpec((BN, 8), lambda i: (i, 0)),
        pl.BlockSpec((H,), lambda i: (0,)),
        pl.BlockSpec((C, H), lambda i: (0, 0)),
        pl.BlockSpec((1, 1, BN), lambda i: (i, 0, 0)),
        pl.BlockSpec((2 * H, 1), lambda i: (0, 0)),
        pl.BlockSpec((1,), lambda i: (0,)),
        pl.BlockSpec((H, NCLS), lambda i: (0, 0)),
        pl.BlockSpec((NCLS,), lambda i: (0,)),
    ],
    out_specs=pl.BlockSpec((BN, NCLS), lambda i: (i, 0)),
    out_shape=jax.ShapeDtypeStruct((N, NCLS), jnp.float32),
)


def kernel(node_features, node_adj, comm_features, comm_adj, node_to_comm_map,
           W1, b1, W2, b2, gate_W, gate_b, cls_W, cls_b):
    del comm_adj
    ones_c = jnp.ones((KE, H), jnp.float32)
    zeros_c = jnp.zeros((328, H), jnp.float32)
    map3 = node_to_comm_map.reshape(GRID, 1, BN)
    src = node_adj[0]
    dst = node_adj[1]

    deg_p = _deg_kernel(dst, ones_c, zeros_c)
    xs1, dinv8 = _tcA(node_features, W1, deg_p)
    acc1 = _segsum_kernel(xs1, src, dst)
    xs2 = _tcB(acc1, dinv8, b1, W2)
    acc2 = _segsum_kernel(xs2, src, dst)
    out = _tcC(acc2, dinv8, b2, comm_features, map3,
               gate_W, gate_b, cls_W, cls_b)
    return out


# speedup vs baseline: 11.5492x; 1.7488x over previous
import functools

import jax
import jax.numpy as jnp
from jax import lax
from jax.experimental import pallas as pl
from jax.experimental.pallas import tpu as pltpu
from jax.experimental.pallas import tpu_sc as plsc

N = 10000
E = 320000
C = 1000
H = 128
NCLS = 40

NSC = 2
NTILE = 16
NP = 10240
SHALF = NP // 2
SACC = SHALF + 8
TRASH = SHALF
SRPT = SHALF // NTILE
EPT = E // NTILE
KE = 80
NCHUNK = EPT // KE
BN = 1000
GRID = N // BN

_MESH = plsc.VectorSubcoreMesh(core_axis_name="c", subcore_axis_name="s",
                               num_cores=NSC, num_subcores=NTILE)


def _localize(dst_v, base, loc_v, lo):
    for j in range(KE // 16):
        d = dst_v[pl.ds(base + j * 16, 16)]
        rel = d - lo
        owned = (rel >= 0) & (rel < SHALF)
        loc_v[pl.ds(j * 16, 16)] = jnp.where(owned, rel, TRASH)


@functools.partial(
    pl.kernel,
    out_type=jax.ShapeDtypeStruct((NP, H), jnp.float32),
    mesh=_MESH,
    scratch_types=[
        pltpu.VMEM((KE,), jnp.int32),
        pltpu.VMEM((KE,), jnp.int32),
        pltpu.VMEM((KE, H), jnp.float32),
        pltpu.VMEM((328, H), jnp.float32),
        pltpu.VMEM_SHARED((SACC, H), jnp.float32),
    ],
)
def _deg_kernel(dst_hbm, ones_hbm, zeros_hbm, out_hbm,
                dst_v, loc_v, ones_v, strip_v, deg_sh):
    cid = lax.axis_index("c")
    sid = lax.axis_index("s")
    lo = cid * SHALF
    r0 = sid * SRPT
    pltpu.sync_copy(zeros_hbm, strip_v)
    pltpu.sync_copy(strip_v, deg_sh.at[pl.ds(r0, 328)])
    pltpu.sync_copy(ones_hbm, ones_v)
    ebase = sid * EPT
    plsc.subcore_barrier()

    def body(i, carry):
        pltpu.sync_copy(dst_hbm.at[pl.ds(ebase + i * KE, KE)], dst_v)
        _localize(dst_v, 0, loc_v, lo)
        pltpu.sync_copy(ones_v, deg_sh.at[loc_v], add=True)
        return carry

    lax.fori_loop(0, NCHUNK, body, 0)
    plsc.subcore_barrier()
    pltpu.sync_copy(deg_sh.at[pl.ds(r0, SRPT)], strip_v.at[pl.ds(0, SRPT)])
    pltpu.sync_copy(strip_v.at[pl.ds(0, SRPT)],
                    out_hbm.at[pl.ds(lo + r0, SRPT)])


@functools.partial(
    pl.kernel,
    out_type=jax.ShapeDtypeStruct((NP, H), jnp.float32),
    mesh=_MESH,
    scratch_types=[
        pltpu.VMEM((KE,), jnp.int32),
        pltpu.VMEM((KE,), jnp.int32),
        pltpu.VMEM((KE,), jnp.int32),
        pltpu.VMEM((KE,), jnp.int32),
        pltpu.VMEM((KE,), jnp.int32),
        pltpu.VMEM((KE,), jnp.int32),
        pltpu.VMEM((KE, H), jnp.float32),
        pltpu.VMEM((KE, H), jnp.float32),
        pltpu.VMEM((SRPT, H), jnp.float32),
        pltpu.VMEM_SHARED((SACC, H), jnp.float32),
        pltpu.SemaphoreType.DMA,
        pltpu.SemaphoreType.DMA,
        pltpu.SemaphoreType.DMA,
        pltpu.SemaphoreType.DMA,
    ],
)
def _segsum_kernel(xs_hbm, src_hbm, dst_hbm, out_hbm,
                   src_a, src_b, dst_a, dst_b, loc_a, loc_b, rows_a, rows_b,
                   strip_v, acc_sh, sem_ia, sem_ib, sem_a, sem_b):
    cid = lax.axis_index("c")
    sid = lax.axis_index("s")
    lo = cid * SHALF
    r0 = sid * SRPT
    pltpu.sync_copy(xs_hbm.at[pl.ds(lo + r0, SRPT)], strip_v)
    pltpu.sync_copy(strip_v, acc_sh.at[pl.ds(r0, SRPT)])
    plsc.subcore_barrier()
    ebase = sid * EPT

    def idx_load(b, sv, dv, sem):
        pltpu.async_copy(src_hbm.at[pl.ds(b, KE)], sv, sem)
        pltpu.async_copy(dst_hbm.at[pl.ds(b, KE)], dv, sem)

    def idx_wait(sv, dv, sem):
        pltpu.make_async_copy(src_hbm.at[pl.ds(0, KE)], sv, sem).wait()
        pltpu.make_async_copy(dst_hbm.at[pl.ds(0, KE)], dv, sem).wait()

    idx_load(ebase, src_a, dst_a, sem_ia)
    idx_wait(src_a, dst_a, sem_ia)
    pltpu.async_copy(xs_hbm.at[src_a], rows_a, sem_a)
    idx_load(ebase + KE, src_b, dst_b, sem_ib)

    def body(t, carry):
        b0 = ebase + 2 * t * KE

        idx_wait(src_b, dst_b, sem_ib)
        cp_b = pltpu.async_copy(xs_hbm.at[src_b], rows_b, sem_b)
        _localize(dst_a, 0, loc_a, lo)
        pltpu.make_async_copy(xs_hbm.at[src_a], rows_a, sem_a).wait()
        pltpu.sync_copy(rows_a, acc_sh.at[loc_a], add=True)

        @pl.when(t < NCHUNK // 2 - 1)
        def _():
            idx_load(b0 + 2 * KE, src_a, dst_a, sem_ia)

        _localize(dst_b, 0, loc_b, lo)
        cp_b.wait()

        @pl.when(t < NCHUNK // 2 - 1)
        def _():
            idx_wait(src_a, dst_a, sem_ia)
            pltpu.async_copy(xs_hbm.at[src_a], rows_a, sem_a)
            idx_load(b0 + 3 * KE, src_b, dst_b, sem_ib)

        pltpu.sync_copy(rows_b, acc_sh.at[loc_b], add=True)
        return carry

    lax.fori_loop(0, NCHUNK // 2, body, 0)
    plsc.subcore_barrier()
    pltpu.sync_copy(acc_sh.at[pl.ds(r0, SRPT)], strip_v)
    pltpu.sync_copy(strip_v, out_hbm.at[pl.ds(lo + r0, SRPT)])


def _tcA_body(x_ref, w1_ref, degp_ref, xs1_ref, dinv8_ref):
    deg = degp_ref[:, 0:1] + 1.0
    dinv = lax.rsqrt(deg)
    xw = jnp.dot(x_ref[...], w1_ref[...], preferred_element_type=jnp.float32)
    xs1_ref[...] = dinv * xw
    dinv8_ref[...] = jnp.broadcast_to(dinv, (BN, 8))


_tcA = pl.pallas_call(
    _tcA_body,
    grid=(GRID,),
    in_specs=[
        pl.BlockSpec((BN, H), lambda i: (i, 0)),
        pl.BlockSpec((H, H), lambda i: (0, 0)),
        pl.BlockSpec((BN, H), lambda i: (i, 0)),
    ],
    out_specs=[
        pl.BlockSpec((BN, H), lambda i: (i, 0)),
        pl.BlockSpec((BN, 8), lambda i: (i, 0)),
    ],
    out_shape=[
        jax.ShapeDtypeStruct((NP, H), jnp.float32),
        jax.ShapeDtypeStruct((N, 8), jnp.float32),
    ],
)


def _tcB_body(acc_ref, dinv8_ref, b1_ref, w2_ref, xs2_ref):
    dinv = dinv8_ref[:, 0:1]
    h1 = jnp.maximum(dinv * acc_ref[...] + b1_ref[...], 0.0)
    xw2 = jnp.dot(h1, w2_ref[...], preferred_element_type=jnp.float32)
    xs2_ref[...] = dinv * xw2


_tcB = pl.pallas_call(
    _tcB_body,
    grid=(GRID,),
    in_specs=[
        pl.BlockSpec((BN, H), lambda i: (i, 0)),
        pl.BlockSpec((BN, 8), lambda i: (i, 0)),
        pl.BlockSpec((H,), lambda i: (0,)),
        pl.BlockSpec((H, H), lambda i: (0, 0)),
    ],
    out_specs=pl.BlockSpec((BN, H), lambda i: (i, 0)),
    out_shape=jax.ShapeDtypeStruct((NP, H), jnp.float32),
)


def _tcC_body(acc_ref, dinv8_ref, b2_ref, comm_ref, map_ref,
              gw_ref, gb_ref, cw_ref, cb_ref, out_ref):
    dinv = dinv8_ref[:, 0:1]
    h2 = jnp.maximum(dinv * acc_ref[...] + b2_ref[...], 0.0)
    m = map_ref[0, 0, :]
    onehot = (m[:, None] == lax.broadcasted_iota(jnp.int32, (1, C), 1)
              ).astype(jnp.float32)
    hc = jnp.dot(onehot, comm_ref[...], preferred_element_type=jnp.float32)
    gate = jax.nn.sigmoid(
        jnp.dot(h2, gw_ref[0:H, :], preferred_element_type=jnp.float32)
        + jnp.dot(hc, gw_ref[H:2 * H, :], preferred_element_type=jnp.float32)
        + gb_ref[...])
    hf = gate * h2 + (1.0 - gate) * hc
    logits = jnp.dot(hf, cw_ref[...], preferred_element_type=jnp.float32) \
        + cb_ref[...]
    mx = jnp.max(logits, axis=-1, keepdims=True)
    z = logits - mx
    lse = jnp.log(jnp.sum(jnp.exp(z), axis=-1, keepdims=True))
    out_ref[...] = z - lse


_tcC = pl.pallas_call(
    _tcC_body,
    grid=(GRID,),
    in_specs=[
        pl.BlockSpec((BN, H), lambda i: (i, 0)),
        pl.BlockSpec((BN, 8), lambda i: (i, 0)),
        pl.BlockSpec((H,), lambda i: (0,)),
        pl.BlockSpec((C, H), lambda i: (0, 0)),
        pl.BlockSpec((1, 1, BN), lambda i: (i, 0, 0)),
        pl.BlockSpec((2 * H, 1), lambda i: (0, 0)),
        pl.BlockSpec((1,), lambda i: (0,)),
        pl.BlockSpec((H, NCLS), lambda i: (0, 0)),
        pl.BlockSpec((NCLS,), lambda i: (0,)),
    ],
    out_specs=pl.BlockSpec((BN, NCLS), lambda i: (i, 0)),
    out_shape=jax.ShapeDtypeStruct((N, NCLS), jnp.float32),
)


def kernel(node_features, node_adj, comm_features, comm_adj, node_to_comm_map,
           W1, b1, W2, b2, gate_W, gate_b, cls_W, cls_b):
    del comm_adj
    ones_c = jnp.ones((KE, H), jnp.float32)
    zeros_c = jnp.zeros((328, H), jnp.float32)
    map3 = node_to_comm_map.reshape(GRID, 1, BN)
    src = node_adj[0]
    dst = node_adj[1]

    deg_p = _deg_kernel(dst, ones_c, zeros_c)
    xs1, dinv8 = _tcA(node_features, W1, deg_p)
    acc1 = _segsum_kernel(xs1, src, dst)
    xs2 = _tcB(acc1, dinv8, b1, W2)
    acc2 = _segsum_kernel(xs2, src, dst)
    out = _tcC(acc2, dinv8, b2, comm_features, map3,
               gate_W, gate_b, cls_W, cls_b)
    return out


# speedup vs baseline: 12.9802x; 1.1239x over previous
import functools

import jax
import jax.numpy as jnp
from jax import lax
from jax.experimental import pallas as pl
from jax.experimental.pallas import tpu as pltpu
from jax.experimental.pallas import tpu_sc as plsc

N = 10000
E = 320000
C = 1000
H = 128
NCLS = 40

NSC = 2
NTILE = 16
NP = 10240
SHALF = NP // 2
SACC = SHALF + 8
TRASH = SHALF
SRPT = SHALF // NTILE
EPT = E // NTILE
KE = 80
NCHUNK = EPT // KE
BN = 1000
GRID = N // BN

_MESH = plsc.VectorSubcoreMesh(core_axis_name="c", subcore_axis_name="s",
                               num_cores=NSC, num_subcores=NTILE)


def _localize(dst_v, base, loc_v, lo):
    for j in range(KE // 16):
        d = dst_v[pl.ds(base + j * 16, 16)]
        rel = d - lo
        owned = (rel >= 0) & (rel < SHALF)
        loc_v[pl.ds(j * 16, 16)] = jnp.where(owned, rel, TRASH)


EB = 3200
EGRID = E // EB


def _deg_tc_body(dst_ref, out_ref):
    i = pl.program_id(0)
    d = dst_ref[0, 0, :]
    hi = d // 100
    lo = d % 100
    cols = lax.broadcasted_iota(jnp.int32, (1, 100), 1)
    oh_hi = (hi[:, None] == cols).astype(jnp.float32)
    oh_lo = (lo[:, None] == cols).astype(jnp.float32)
    delta = lax.dot_general(oh_hi, oh_lo, (((0,), (0,)), ((), ())),
                            preferred_element_type=jnp.float32)

    @pl.when(i == 0)
    def _():
        out_ref[...] = jnp.zeros_like(out_ref)

    out_ref[...] += delta


_deg_tc = pl.pallas_call(
    _deg_tc_body,
    grid=(EGRID,),
    in_specs=[pl.BlockSpec((1, 1, EB), lambda i: (i, 0, 0))],
    out_specs=pl.BlockSpec((100, 100), lambda i: (0, 0)),
    out_shape=jax.ShapeDtypeStruct((100, 100), jnp.float32),
)


@functools.partial(
    pl.kernel,
    out_type=jax.ShapeDtypeStruct((NP, H), jnp.float32),
    mesh=_MESH,
    scratch_types=[
        pltpu.VMEM((KE,), jnp.int32),
        pltpu.VMEM((KE,), jnp.int32),
        pltpu.VMEM((KE,), jnp.int32),
        pltpu.VMEM((KE,), jnp.int32),
        pltpu.VMEM((KE,), jnp.int32),
        pltpu.VMEM((KE,), jnp.int32),
        pltpu.VMEM((KE, H), jnp.float32),
        pltpu.VMEM((KE, H), jnp.float32),
        pltpu.VMEM((SRPT, H), jnp.float32),
        pltpu.VMEM_SHARED((SACC, H), jnp.float32),
        pltpu.SemaphoreType.DMA,
        pltpu.SemaphoreType.DMA,
        pltpu.SemaphoreType.DMA,
        pltpu.SemaphoreType.DMA,
    ],
)
def _segsum_kernel(xs_hbm, src_hbm, dst_hbm, out_hbm,
                   src_a, src_b, dst_a, dst_b, loc_a, loc_b, rows_a, rows_b,
                   strip_v, acc_sh, sem_ia, sem_ib, sem_a, sem_b):
    cid = lax.axis_index("c")
    sid = lax.axis_index("s")
    lo = cid * SHALF
    r0 = sid * SRPT
    pltpu.sync_copy(xs_hbm.at[pl.ds(lo + r0, SRPT)], strip_v)
    pltpu.sync_copy(strip_v, acc_sh.at[pl.ds(r0, SRPT)])
    plsc.subcore_barrier()
    ebase = sid * EPT

    def idx_load(b, sv, dv, sem):
        pltpu.async_copy(src_hbm.at[pl.ds(b, KE)], sv, sem)
        pltpu.async_copy(dst_hbm.at[pl.ds(b, KE)], dv, sem)

    def idx_wait(sv, dv, sem):
        pltpu.make_async_copy(src_hbm.at[pl.ds(0, KE)], sv, sem).wait()
        pltpu.make_async_copy(dst_hbm.at[pl.ds(0, KE)], dv, sem).wait()

    idx_load(ebase, src_a, dst_a, sem_ia)
    idx_wait(src_a, dst_a, sem_ia)
    pltpu.async_copy(xs_hbm.at[src_a], rows_a, sem_a)
    idx_load(ebase + KE, src_b, dst_b, sem_ib)

    def body(t, carry):
        b0 = ebase + 2 * t * KE

        idx_wait(src_b, dst_b, sem_ib)
        cp_b = pltpu.async_copy(xs_hbm.at[src_b], rows_b, sem_b)
        _localize(dst_a, 0, loc_a, lo)
        pltpu.make_async_copy(xs_hbm.at[src_a], rows_a, sem_a).wait()
        pltpu.sync_copy(rows_a, acc_sh.at[loc_a], add=True)

        @pl.when(t < NCHUNK // 2 - 1)
        def _():
            idx_load(b0 + 2 * KE, src_a, dst_a, sem_ia)

        _localize(dst_b, 0, loc_b, lo)
        cp_b.wait()

        @pl.when(t < NCHUNK // 2 - 1)
        def _():
            idx_wait(src_a, dst_a, sem_ia)
            pltpu.async_copy(xs_hbm.at[src_a], rows_a, sem_a)
            idx_load(b0 + 3 * KE, src_b, dst_b, sem_ib)

        pltpu.sync_copy(rows_b, acc_sh.at[loc_b], add=True)
        return carry

    lax.fori_loop(0, NCHUNK // 2, body, 0)
    plsc.subcore_barrier()
    pltpu.sync_copy(acc_sh.at[pl.ds(r0, SRPT)], strip_v)
    pltpu.sync_copy(strip_v, out_hbm.at[pl.ds(lo + r0, SRPT)])


def _tcA_body(x_ref, w1_ref, deg_ref, xs1_ref, dinv8_ref):
    deg = deg_ref[...] + 1.0
    dinv = lax.rsqrt(deg)
    xw = jnp.dot(x_ref[...], w1_ref[...], preferred_element_type=jnp.float32)
    xs1_ref[...] = dinv * xw
    dinv8_ref[...] = jnp.broadcast_to(dinv, (BN, 8))


_tcA = pl.pallas_call(
    _tcA_body,
    grid=(GRID,),
    in_specs=[
        pl.BlockSpec((BN, H), lambda i: (i, 0)),
        pl.BlockSpec((H, H), lambda i: (0, 0)),
        pl.BlockSpec((BN, 1), lambda i: (i, 0)),
    ],
    out_specs=[
        pl.BlockSpec((BN, H), lambda i: (i, 0)),
        pl.BlockSpec((BN, 8), lambda i: (i, 0)),
    ],
    out_shape=[
        jax.ShapeDtypeStruct((NP, H), jnp.float32),
        jax.ShapeDtypeStruct((N, 8), jnp.float32),
    ],
)


def _tcB_body(acc_ref, dinv8_ref, b1_ref, w2_ref, xs2_ref):
    dinv = dinv8_ref[:, 0:1]
    h1 = jnp.maximum(dinv * acc_ref[...] + b1_ref[...], 0.0)
    xw2 = jnp.dot(h1, w2_ref[...], preferred_element_type=jnp.float32)
    xs2_ref[...] = dinv * xw2


_tcB = pl.pallas_call(
    _tcB_body,
    grid=(GRID,),
    in_specs=[
        pl.BlockSpec((BN, H), lambda i: (i, 0)),
        pl.BlockSpec((BN, 8), lambda i: (i, 0)),
        pl.BlockSpec((H,), lambda i: (0,)),
        pl.BlockSpec((H, H), lambda i: (0, 0)),
    ],
    out_specs=pl.BlockSpec((BN, H), lambda i: (i, 0)),
    out_shape=jax.ShapeDtypeStruct((NP, H), jnp.float32),
)


def _tcC_body(acc_ref, dinv8_ref, b2_ref, comm_ref, map_ref,
              gw_ref, gb_ref, cw_ref, cb_ref, out_ref):
    dinv = dinv8_ref[:, 0:1]
    h2 = jnp.maximum(dinv * acc_ref[...] + b2_ref[...], 0.0)
    m = map_ref[0, 0, :]
    onehot = (m[:, None] == lax.broadcasted_iota(jnp.int32, (1, C), 1)
              ).astype(jnp.float32)
    hc = jnp.dot(onehot, comm_ref[...], preferred_element_type=jnp.float32)
    gate = jax.nn.sigmoid(
        jnp.dot(h2, gw_ref[0:H, :], preferred_element_type=jnp.float32)
        + jnp.dot(hc, gw_ref[H:2 * H, :], preferred_element_type=jnp.float32)
        + gb_ref[...])
    hf = gate * h2 + (1.0 - gate) * hc
    logits = jnp.dot(hf, cw_ref[...], preferred_element_type=jnp.float32) \
        + cb_ref[...]
    mx = jnp.max(logits, axis=-1, keepdims=True)
    z = logits - mx
    lse = jnp.log(jnp.sum(jnp.exp(z), axis=-1, keepdims=True))
    out_ref[...] = z - lse


_tcC = pl.pallas_call(
    _tcC_body,
    grid=(GRID,),
    in_specs=[
        pl.BlockSpec((BN, H), lambda i: (i, 0)),
        pl.BlockSpec((BN, 8), lambda i: (i, 0)),
        pl.BlockSpec((H,), lambda i: (0,)),
        pl.BlockSpec((C, H), lambda i: (0, 0)),
        pl.BlockSpec((1, 1, BN), lambda i: (i, 0, 0)),
        pl.BlockSpec((2 * H, 1), lambda i: (0, 0)),
        pl.BlockSpec((1,), lambda i: (0,)),
        pl.BlockSpec((H, NCLS), lambda i: (0, 0)),
        pl.BlockSpec((NCLS,), lambda i: (0,)),
    ],
    out_specs=pl.BlockSpec((BN, NCLS), lambda i: (i, 0)),
    out_shape=jax.ShapeDtypeStruct((N, NCLS), jnp.float32),
)


def kernel(node_features, node_adj, comm_features, comm_adj, node_to_comm_map,
           W1, b1, W2, b2, gate_W, gate_b, cls_W, cls_b):
    del comm_adj
    map3 = node_to_comm_map.reshape(GRID, 1, BN)
    src = node_adj[0]
    dst = node_adj[1]

    deg_col = _deg_tc(dst.reshape(EGRID, 1, EB)).reshape(N, 1)
    xs1, dinv8 = _tcA(node_features, W1, deg_col)
    acc1 = _segsum_kernel(xs1, src, dst)
    xs2 = _tcB(acc1, dinv8, b1, W2)
    acc2 = _segsum_kernel(xs2, src, dst)
    out = _tcC(acc2, dinv8, b2, comm_features, map3,
               gate_W, gate_b, cls_W, cls_b)
    return out


# speedup vs baseline: 13.3390x; 1.0276x over previous
import functools

import jax
import jax.numpy as jnp
from jax import lax
from jax.experimental import pallas as pl
from jax.experimental.pallas import tpu as pltpu
from jax.experimental.pallas import tpu_sc as plsc

N = 10000
E = 320000
C = 1000
H = 128
NCLS = 40

NSC = 2
NTILE = 16
NP = 10240
SHALF = NP // 2
SACC = SHALF + 128
TRASH = SHALF
SRPT = SHALF // NTILE
EPT = E // NTILE
KE = 80
NCHUNK = EPT // KE
BN = 1000
GRID = N // BN

_MESH = plsc.VectorSubcoreMesh(core_axis_name="c", subcore_axis_name="s",
                               num_cores=NSC, num_subcores=NTILE)


def _localize(dst_v, loc_v, lo, trash):
    for j in range(KE // 16):
        d = dst_v[pl.ds(j * 16, 16)]
        rel = d - lo
        owned = (rel >= 0) & (rel < SHALF)
        loc_v[pl.ds(j * 16, 16)] = jnp.where(owned, rel, trash + (j % 8))


EB = 3200
EGRID = E // EB


def _deg_tc_body(dst_ref, out_ref):
    i = pl.program_id(0)
    d = dst_ref[0, 0, :]
    hi = lax.shift_right_logical(d * 41944, 22)
    lo = d - hi * 100
    cols = lax.broadcasted_iota(jnp.int32, (1, 100), 1)
    oh_hi = (hi[:, None] == cols).astype(jnp.float32)
    oh_lo = (lo[:, None] == cols).astype(jnp.float32)
    delta = lax.dot_general(oh_hi, oh_lo, (((0,), (0,)), ((), ())),
                            preferred_element_type=jnp.float32)

    @pl.when(i == 0)
    def _():
        out_ref[...] = jnp.zeros_like(out_ref)

    out_ref[...] += delta


_deg_tc = pl.pallas_call(
    _deg_tc_body,
    grid=(EGRID,),
    in_specs=[pl.BlockSpec((1, 1, EB), lambda i: (i, 0, 0))],
    out_specs=pl.BlockSpec((100, 100), lambda i: (0, 0)),
    out_shape=jax.ShapeDtypeStruct((100, 100), jnp.float32),
)


@functools.partial(
    pl.kernel,
    out_type=jax.ShapeDtypeStruct((NP, H), jnp.float32),
    mesh=_MESH,
    scratch_types=[
        pltpu.VMEM((KE,), jnp.int32),
        pltpu.VMEM((KE,), jnp.int32),
        pltpu.VMEM((KE,), jnp.int32),
        pltpu.VMEM((KE,), jnp.int32),
        pltpu.VMEM((KE,), jnp.int32),
        pltpu.VMEM((KE,), jnp.int32),
        pltpu.VMEM((KE, H), jnp.float32),
        pltpu.VMEM((KE, H), jnp.float32),
        pltpu.VMEM((SRPT, H), jnp.float32),
        pltpu.VMEM_SHARED((SACC, H), jnp.float32),
        pltpu.SemaphoreType.DMA,
        pltpu.SemaphoreType.DMA,
        pltpu.SemaphoreType.DMA,
        pltpu.SemaphoreType.DMA,
    ],
)
def _segsum_kernel(xs_hbm, src_hbm, dst_hbm, out_hbm,
                   src_a, src_b, dst_a, dst_b, loc_a, loc_b, rows_a, rows_b,
                   strip_v, acc_sh, sem_ia, sem_ib, sem_a, sem_b):
    cid = lax.axis_index("c")
    sid = lax.axis_index("s")
    lo = cid * SHALF
    trash = TRASH + sid * 8
    r0 = sid * SRPT
    pltpu.sync_copy(xs_hbm.at[pl.ds(lo + r0, SRPT)], strip_v)
    pltpu.sync_copy(strip_v, acc_sh.at[pl.ds(r0, SRPT)])
    plsc.subcore_barrier()
    ebase = sid * EPT

    def idx_load(b, sv, dv, sem):
        pltpu.async_copy(src_hbm.at[pl.ds(b, KE)], sv, sem)
        pltpu.async_copy(dst_hbm.at[pl.ds(b, KE)], dv, sem)

    def idx_wait(sv, dv, sem):
        pltpu.make_async_copy(src_hbm.at[pl.ds(0, KE)], sv, sem).wait()
        pltpu.make_async_copy(dst_hbm.at[pl.ds(0, KE)], dv, sem).wait()

    idx_load(ebase, src_a, dst_a, sem_ia)
    idx_wait(src_a, dst_a, sem_ia)
    pltpu.async_copy(xs_hbm.at[src_a], rows_a, sem_a)
    idx_load(ebase + KE, src_b, dst_b, sem_ib)

    def body(t, carry):
        b0 = ebase + 2 * t * KE

        idx_wait(src_b, dst_b, sem_ib)
        cp_b = pltpu.async_copy(xs_hbm.at[src_b], rows_b, sem_b)
        _localize(dst_a, loc_a, lo, trash)
        pltpu.make_async_copy(xs_hbm.at[src_a], rows_a, sem_a).wait()
        pltpu.sync_copy(rows_a, acc_sh.at[loc_a], add=True)

        @pl.when(t < NCHUNK // 2 - 1)
        def _():
            idx_load(b0 + 2 * KE, src_a, dst_a, sem_ia)

        _localize(dst_b, loc_b, lo, trash)
        cp_b.wait()

        @pl.when(t < NCHUNK // 2 - 1)
        def _():
            idx_wait(src_a, dst_a, sem_ia)
            pltpu.async_copy(xs_hbm.at[src_a], rows_a, sem_a)
            idx_load(b0 + 3 * KE, src_b, dst_b, sem_ib)

        pltpu.sync_copy(rows_b, acc_sh.at[loc_b], add=True)
        return carry

    lax.fori_loop(0, NCHUNK // 2, body, 0)
    plsc.subcore_barrier()
    pltpu.sync_copy(acc_sh.at[pl.ds(r0, SRPT)], strip_v)
    pltpu.sync_copy(strip_v, out_hbm.at[pl.ds(lo + r0, SRPT)])


def _tcA_body(x_ref, w1_ref, deg_ref, xs1_ref, dinv8_ref):
    deg = deg_ref[...] + 1.0
    dinv = lax.rsqrt(deg)
    xw = jnp.dot(x_ref[...], w1_ref[...], preferred_element_type=jnp.float32)
    xs1_ref[...] = dinv * xw
    dinv8_ref[...] = jnp.broadcast_to(dinv, (BN, 8))


_tcA = pl.pallas_call(
    _tcA_body,
    grid=(GRID,),
    in_specs=[
        pl.BlockSpec((BN, H), lambda i: (i, 0)),
        pl.BlockSpec((H, H), lambda i: (0, 0)),
        pl.BlockSpec((BN, 1), lambda i: (i, 0)),
    ],
    out_specs=[
        pl.BlockSpec((BN, H), lambda i: (i, 0)),
        pl.BlockSpec((BN, 8), lambda i: (i, 0)),
    ],
    out_shape=[
        jax.ShapeDtypeStruct((NP, H), jnp.float32),
        jax.ShapeDtypeStruct((N, 8), jnp.float32),
    ],
)


def _tcB_body(acc_ref, dinv8_ref, b1_ref, w2_ref, xs2_ref):
    dinv = dinv8_ref[:, 0:1]
    h1 = jnp.maximum(dinv * acc_ref[...] + b1_ref[...], 0.0)
    xw2 = jnp.dot(h1, w2_ref[...], preferred_element_type=jnp.float32)
    xs2_ref[...] = dinv * xw2


_tcB = pl.pallas_call(
    _tcB_body,
    grid=(GRID,),
    in_specs=[
        pl.BlockSpec((BN, H), lambda i: (i, 0)),
        pl.BlockSpec((BN, 8), lambda i: (i, 0)),
        pl.BlockSpec((H,), lambda i: (0,)),
        pl.BlockSpec((H, H), lambda i: (0, 0)),
    ],
    out_specs=pl.BlockSpec((BN, H), lambda i: (i, 0)),
    out_shape=jax.ShapeDtypeStruct((NP, H), jnp.float32),
)


def _tcC_body(acc_ref, dinv8_ref, b2_ref, comm_ref, map_ref,
              gw_ref, gb_ref, cw_ref, cb_ref, out_ref):
    dinv = dinv8_ref[:, 0:1]
    h2 = jnp.maximum(dinv * acc_ref[...] + b2_ref[...], 0.0)
    m = map_ref[0, 0, :]
    onehot = (m[:, None] == lax.broadcasted_iota(jnp.int32, (1, C), 1)
              ).astype(jnp.float32)
    hc = jnp.dot(onehot, comm_ref[...], preferred_element_type=jnp.float32)
    gate = jax.nn.sigmoid(
        jnp.dot(h2, gw_ref[0:H, :], preferred_element_type=jnp.float32)
        + jnp.dot(hc, gw_ref[H:2 * H, :], preferred_element_type=jnp.float32)
        + gb_ref[...])
    hf = gate * h2 + (1.0 - gate) * hc
    logits = jnp.dot(hf, cw_ref[...], preferred_element_type=jnp.float32) \
        + cb_ref[...]
    mx = jnp.max(logits, axis=-1, keepdims=True)
    z = logits - mx
    lse = jnp.log(jnp.sum(jnp.exp(z), axis=-1, keepdims=True))
    out_ref[...] = z - lse


_tcC = pl.pallas_call(
    _tcC_body,
    grid=(GRID,),
    in_specs=[
        pl.BlockSpec((BN, H), lambda i: (i, 0)),
        pl.BlockSpec((BN, 8), lambda i: (i, 0)),
        pl.BlockSpec((H,), lambda i: (0,)),
        pl.BlockSpec((C, H), lambda i: (0, 0)),
        pl.BlockSpec((1, 1, BN), lambda i: (i, 0, 0)),
        pl.BlockSpec((2 * H, 1), lambda i: (0, 0)),
        pl.BlockSpec((1,), lambda i: (0,)),
        pl.BlockSpec((H, NCLS), lambda i: (0, 0)),
        pl.BlockSpec((NCLS,), lambda i: (0,)),
    ],
    out_specs=pl.BlockSpec((BN, NCLS), lambda i: (i, 0)),
    out_shape=jax.ShapeDtypeStruct((N, NCLS), jnp.float32),
)


def kernel(node_features, node_adj, comm_features, comm_adj, node_to_comm_map,
           W1, b1, W2, b2, gate_W, gate_b, cls_W, cls_b):
    del comm_adj
    map3 = node_to_comm_map.reshape(GRID, 1, BN)
    src = node_adj[0]
    dst = node_adj[1]

    deg_col = _deg_tc(dst.reshape(EGRID, 1, EB)).reshape(N, 1)
    xs1, dinv8 = _tcA(node_features, W1, deg_col)
    acc1 = _segsum_kernel(xs1, src, dst)
    xs2 = _tcB(acc1, dinv8, b1, W2)
    acc2 = _segsum_kernel(xs2, src, dst)
    out = _tcC(acc2, dinv8, b2, comm_features, map3,
               gate_W, gate_b, cls_W, cls_b)
    return out


# speedup vs baseline: 14.5501x; 1.0908x over previous
import functools

import jax
import jax.numpy as jnp
from jax import lax
from jax.experimental import pallas as pl
from jax.experimental.pallas import tpu as pltpu
from jax.experimental.pallas import tpu_sc as plsc

N = 10000
E = 320000
C = 1000
H = 128
NCLS = 40

NSC = 2
NTILE = 16
NP = 10240
SHALF = NP // 2
SACC = SHALF + 128
TRASH = SHALF
SRPT = SHALF // NTILE
EPT = E // NTILE
KE = 128
KT = 32
NFULL = (EPT - KT) // KE
NCHUNK = NFULL
BN = 1000
GRID = N // BN

_MESH = plsc.VectorSubcoreMesh(core_axis_name="c", subcore_axis_name="s",
                               num_cores=NSC, num_subcores=NTILE)


def _localize(dst_v, loc_v, lo, trash, n=KE):
    for j in range(n // 16):
        d = dst_v[pl.ds(j * 16, 16)]
        rel = d - lo
        owned = (rel >= 0) & (rel < SHALF)
        loc_v[pl.ds(j * 16, 16)] = jnp.where(owned, rel, trash + (j % 8))


EB = 3200
EGRID = E // EB


def _deg_tc_body(dst_ref, out_ref):
    i = pl.program_id(0)
    d = dst_ref[0, 0, :]
    hi = lax.shift_right_logical(d * 41944, 22)
    lo = d - hi * 100
    cols = lax.broadcasted_iota(jnp.int32, (1, 100), 1)
    oh_hi = (hi[:, None] == cols).astype(jnp.float32)
    oh_lo = (lo[:, None] == cols).astype(jnp.float32)
    delta = lax.dot_general(oh_hi, oh_lo, (((0,), (0,)), ((), ())),
                            preferred_element_type=jnp.float32)

    @pl.when(i == 0)
    def _():
        out_ref[...] = jnp.zeros_like(out_ref)

    out_ref[...] += delta


_deg_tc = pl.pallas_call(
    _deg_tc_body,
    grid=(EGRID,),
    in_specs=[pl.BlockSpec((1, 1, EB), lambda i: (i, 0, 0))],
    out_specs=pl.BlockSpec((100, 100), lambda i: (0, 0)),
    out_shape=jax.ShapeDtypeStruct((100, 100), jnp.float32),
)


@functools.partial(
    pl.kernel,
    out_type=jax.ShapeDtypeStruct((NP, H), jnp.float32),
    mesh=_MESH,
    scratch_types=[
        pltpu.VMEM((KE,), jnp.int32),
        pltpu.VMEM((KE,), jnp.int32),
        pltpu.VMEM((KE,), jnp.int32),
        pltpu.VMEM((KE,), jnp.int32),
        pltpu.VMEM((KE,), jnp.int32),
        pltpu.VMEM((KE,), jnp.int32),
        pltpu.VMEM((KT,), jnp.int32),
        pltpu.VMEM((KE, H), jnp.float32),
        pltpu.VMEM((KE, H), jnp.float32),
        pltpu.VMEM((SRPT, H), jnp.float32),
        pltpu.VMEM_SHARED((SACC, H), jnp.float32),
        pltpu.SemaphoreType.DMA,
        pltpu.SemaphoreType.DMA,
        pltpu.SemaphoreType.DMA,
        pltpu.SemaphoreType.DMA,
    ],
)
def _segsum_kernel(xs_hbm, src_hbm, dst_hbm, out_hbm,
                   src_a, src_b, dst_a, dst_b, loc_a, loc_b, loc_t,
                   rows_a, rows_b, strip_v, acc_sh,
                   sem_ia, sem_ib, sem_a, sem_b):
    cid = lax.axis_index("c")
    sid = lax.axis_index("s")
    lo = cid * SHALF
    trash = TRASH + sid * 8
    r0 = sid * SRPT
    pltpu.sync_copy(xs_hbm.at[pl.ds(lo + r0, SRPT)], strip_v)
    pltpu.sync_copy(strip_v, acc_sh.at[pl.ds(r0, SRPT)])
    plsc.subcore_barrier()
    ebase = sid * EPT

    def idx_load(b, sv, dv, sem):
        pltpu.async_copy(src_hbm.at[pl.ds(b, KE)], sv, sem)
        pltpu.async_copy(dst_hbm.at[pl.ds(b, KE)], dv, sem)

    def idx_wait(sv, dv, sem):
        pltpu.make_async_copy(src_hbm.at[pl.ds(0, KE)], sv, sem).wait()
        pltpu.make_async_copy(dst_hbm.at[pl.ds(0, KE)], dv, sem).wait()

    idx_load(ebase, src_a, dst_a, sem_ia)
    idx_wait(src_a, dst_a, sem_ia)
    pltpu.async_copy(xs_hbm.at[src_a], rows_a, sem_a)
    idx_load(ebase + KE, src_b, dst_b, sem_ib)

    def body(t, carry):
        b0 = ebase + 2 * t * KE

        idx_wait(src_b, dst_b, sem_ib)
        cp_b = pltpu.async_copy(xs_hbm.at[src_b], rows_b, sem_b)
        _localize(dst_a, loc_a, lo, trash)
        pltpu.make_async_copy(xs_hbm.at[src_a], rows_a, sem_a).wait()
        pltpu.sync_copy(rows_a, acc_sh.at[loc_a], add=True)

        @pl.when(t < NCHUNK // 2 - 1)
        def _():
            idx_load(b0 + 2 * KE, src_a, dst_a, sem_ia)

        _localize(dst_b, loc_b, lo, trash)
        cp_b.wait()

        @pl.when(t < NCHUNK // 2 - 1)
        def _():
            idx_wait(src_a, dst_a, sem_ia)
            pltpu.async_copy(xs_hbm.at[src_a], rows_a, sem_a)
            idx_load(b0 + 3 * KE, src_b, dst_b, sem_ib)

        pltpu.sync_copy(rows_b, acc_sh.at[loc_b], add=True)
        return carry

    lax.fori_loop(0, NCHUNK // 2, body, 0)

    bt = ebase + NFULL * KE
    pltpu.sync_copy(src_hbm.at[pl.ds(bt, KT)], src_a.at[pl.ds(0, KT)])
    pltpu.sync_copy(dst_hbm.at[pl.ds(bt, KT)], dst_a.at[pl.ds(0, KT)])
    _localize(dst_a, loc_t, lo, trash, n=KT)
    pltpu.async_copy(xs_hbm.at[src_a.at[pl.ds(0, KT)]],
                     rows_a.at[pl.ds(0, KT)], sem_a).wait()
    pltpu.sync_copy(rows_a.at[pl.ds(0, KT)], acc_sh.at[loc_t], add=True)

    plsc.subcore_barrier()
    pltpu.sync_copy(acc_sh.at[pl.ds(r0, SRPT)], strip_v)
    pltpu.sync_copy(strip_v, out_hbm.at[pl.ds(lo + r0, SRPT)])


def _tcA_body(x_ref, w1_ref, deg_ref, xs1_ref, dinv8_ref):
    deg = deg_ref[...] + 1.0
    dinv = lax.rsqrt(deg)
    xw = jnp.dot(x_ref[...], w1_ref[...], preferred_element_type=jnp.float32)
    xs1_ref[...] = dinv * xw
    dinv8_ref[...] = jnp.broadcast_to(dinv, (BN, 8))


_tcA = pl.pallas_call(
    _tcA_body,
    grid=(GRID,),
    in_specs=[
        pl.BlockSpec((BN, H), lambda i: (i, 0)),
        pl.BlockSpec((H, H), lambda i: (0, 0)),
        pl.BlockSpec((BN, 1), lambda i: (i, 0)),
    ],
    out_specs=[
        pl.BlockSpec((BN, H), lambda i: (i, 0)),
        pl.BlockSpec((BN, 8), lambda i: (i, 0)),
    ],
    out_shape=[
        jax.ShapeDtypeStruct((NP, H), jnp.float32),
        jax.ShapeDtypeStruct((N, 8), jnp.float32),
    ],
)


def _tcB_body(acc_ref, dinv8_ref, b1_ref, w2_ref, xs2_ref):
    dinv = dinv8_ref[:, 0:1]
    h1 = jnp.maximum(dinv * acc_ref[...] + b1_ref[...], 0.0)
    xw2 = jnp.dot(h1, w2_ref[...], preferred_element_type=jnp.float32)
    xs2_ref[...] = dinv * xw2


_tcB = pl.pallas_call(
    _tcB_body,
    grid=(GRID,),
    in_specs=[
        pl.BlockSpec((BN, H), lambda i: (i, 0)),
        pl.BlockSpec((BN, 8), lambda i: (i, 0)),
        pl.BlockSpec((H,), lambda i: (0,)),
        pl.BlockSpec((H, H), lambda i: (0, 0)),
    ],
    out_specs=pl.BlockSpec((BN, H), lambda i: (i, 0)),
    out_shape=jax.ShapeDtypeStruct((NP, H), jnp.float32),
)


def _tcC_body(acc_ref, dinv8_ref, b2_ref, comm_ref, map_ref,
              gw_ref, gb_ref, cw_ref, cb_ref, out_ref):
    dinv = dinv8_ref[:, 0:1]
    h2 = jnp.maximum(dinv * acc_ref[...] + b2_ref[...], 0.0)
    m = map_ref[0, 0, :]
    onehot = (m[:, None] == lax.broadcasted_iota(jnp.int32, (1, C), 1)
              ).astype(jnp.float32)
    hc = jnp.dot(onehot, comm_ref[...], preferred_element_type=jnp.float32)
    gate = jax.nn.sigmoid(
        jnp.dot(h2, gw_ref[0:H, :], preferred_element_type=jnp.float32)
        + jnp.dot(hc, gw_ref[H:2 * H, :], preferred_element_type=jnp.float32)
        + gb_ref[...])
    hf = gate * h2 + (1.0 - gate) * hc
    logits = jnp.dot(hf, cw_ref[...], preferred_element_type=jnp.float32) \
        + cb_ref[...]
    mx = jnp.max(logits, axis=-1, keepdims=True)
    z = logits - mx
    lse = jnp.log(jnp.sum(jnp.exp(z), axis=-1, keepdims=True))
    out_ref[...] = z - lse


_tcC = pl.pallas_call(
    _tcC_body,
    grid=(GRID,),
    in_specs=[
        pl.BlockSpec((BN, H), lambda i: (i, 0)),
        pl.BlockSpec((BN, 8), lambda i: (i, 0)),
        pl.BlockSpec((H,), lambda i: (0,)),
        pl.BlockSpec((C, H), lambda i: (0, 0)),
        pl.BlockSpec((1, 1, BN), lambda i: (i, 0, 0)),
        pl.BlockSpec((2 * H, 1), lambda i: (0, 0)),
        pl.BlockSpec((1,), lambda i: (0,)),
        pl.BlockSpec((H, NCLS), lambda i: (0, 0)),
        pl.BlockSpec((NCLS,), lambda i: (0,)),
    ],
    out_specs=pl.BlockSpec((BN, NCLS), lambda i: (i, 0)),
    out_shape=jax.ShapeDtypeStruct((N, NCLS), jnp.float32),
)


def kernel(node_features, node_adj, comm_features, comm_adj, node_to_comm_map,
           W1, b1, W2, b2, gate_W, gate_b, cls_W, cls_b):
    del comm_adj
    map3 = node_to_comm_map.reshape(GRID, 1, BN)
    src = node_adj[0]
    dst = node_adj[1]

    deg_col = _deg_tc(dst.reshape(EGRID, 1, EB)).reshape(N, 1)
    xs1, dinv8 = _tcA(node_features, W1, deg_col)
    acc1 = _segsum_kernel(xs1, src, dst)
    xs2 = _tcB(acc1, dinv8, b1, W2)
    acc2 = _segsum_kernel(xs2, src, dst)
    out = _tcC(acc2, dinv8, b2, comm_features, map3,
               gate_W, gate_b, cls_W, cls_b)
    return out
